# baseline (speedup 1.0000x reference)
import jax
import jax.numpy as jnp
from jax import lax
from jax.experimental import pallas as pl
from jax.experimental.pallas import tpu as pltpu
from jax.experimental.pallas import tpu_sc as plsc

F = 100000
V = 100000
E = 300000

NC = 2
NS = 16
NW = NC * NS
CHUNK = 512
NB = CHUNK // 128
KCH = 20
PW = CHUNK * KCH
E_PAD = NW * PW
R_PAD = 100352
STRIPE = R_PAD // NS
RB = 1024

_LN2 = 0.6931471805599453


def _log_1_8(s):
    bits = lax.bitcast_convert_type(s, jnp.int32)
    e = jnp.right_shift(bits, 23) - 127
    mb = jnp.bitwise_or(jnp.bitwise_and(bits, 0x7FFFFF), 0x3F800000)
    m = lax.bitcast_convert_type(mb, jnp.float32)
    z = (m - 1.0) / (m + 1.0)
    z2 = z * z
    p = 2.0 * z * (1.0 + z2 * (1.0 / 3.0 + z2 * (0.2 + z2 * (1.0 / 7.0))))
    return e.astype(jnp.float32) * _LN2 + p


def _k1_body(fb_hbm, fidx2_hbm, vidx2_hbm, evi_hbm, m01_hbm, z8_hbm,
             f2v_hbm, vaccp_hbm,
             vacc_sp, fidx_a, vidx_a, e0b, e1b, m00, m01, m10, m11,
             fb0, fb1, oa0, ob0, oa1, ob1, o80, o81,
             sg0, sg1, ss0, ss1, so0, so1):
    cid = lax.axis_index("c")
    sid = lax.axis_index("s")
    pltpu.sync_copy(z8_hbm.at[pl.ds(sid * STRIPE, STRIPE)],
                    vacc_sp.at[pl.ds(sid * STRIPE, STRIPE)])
    pltpu.sync_copy(z8_hbm.at[pl.ds(0, CHUNK)], o80)
    pltpu.sync_copy(z8_hbm.at[pl.ds(0, CHUNK)], o81)
    plsc.subcore_barrier()

    w = cid * NS + sid
    rows_pw = PW // 128
    pltpu.sync_copy(fidx2_hbm.at[pl.ds(w * rows_pw, rows_pw)], fidx_a)
    pltpu.sync_copy(vidx2_hbm.at[pl.ds(w * rows_pw, rows_pw)], vidx_a)

    iota16 = lax.iota(jnp.int32, 16)
    zeros16 = jnp.zeros((16,), jnp.int32)
    ones16 = jnp.full((16,), 1, jnp.int32)
    ebs = [e0b, e1b]
    m0s = [m00, m01]
    m1s = [m10, m11]
    fbs = [fb0, fb1]
    oas = [oa0, oa1]
    obs = [ob0, ob1]
    o8s = [o80, o81]
    sgs = [sg0, sg1]
    sss = [ss0, ss1]
    sso = [so0, so1]

    def fire_gathers(g):
        slot = g % 2
        off = w * PW + g * CHUNK
        descs = [pltpu.async_copy(fb_hbm.at[fidx_a.at[g * NB + j]],
                                  fbs[slot].at[pl.ds(j * 128, 128)], sgs[slot])
                 for j in range(NB)]
        descs.append(pltpu.async_copy(m01_hbm.at[0, pl.ds(off, CHUNK)],
                                      m0s[slot], sgs[slot]))
        descs.append(pltpu.async_copy(m01_hbm.at[1, pl.ds(off, CHUNK)],
                                      m1s[slot], sgs[slot]))
        descs.append(pltpu.async_copy(evi_hbm.at[pl.ds(off, CHUNK)],
                                      ebs[slot], sgs[slot]))
        return descs

    def compute(g):
        slot = g % 2
        fbrows_v, f2v8_v = fbs[slot], o8s[slot]
        m0_v, m1_v, evi_v = m0s[slot], m1s[slot], ebs[slot]
        oa_v, ob_v = oas[slot], obs[slot]

        def grp(i, c2):
            sl = pl.ds(i * 16, 16)
            idxv = iota16 + i * 16
            d = evi_v[sl]
            m0 = m0_v[sl]
            m1 = m1_v[sl]
            B = [plsc.load_gather(fbrows_v,
                                  [idxv, jnp.full((16,), b, jnp.int32)])
                 for b in range(8)]
            M = jnp.maximum(jnp.maximum(jnp.maximum(B[0], B[1]),
                                        jnp.maximum(B[2], B[3])),
                            jnp.maximum(jnp.maximum(B[4], B[5]),
                                        jnp.maximum(B[6], B[7])))
            Eb = [jnp.exp(x - M) for x in B]
            a = Eb[0] + Eb[1]
            b2 = Eb[2] + Eb[3]
            c = Eb[4] + Eb[5]
            f = Eb[6] + Eb[7]
            g0 = Eb[0] + Eb[2]
            h0 = Eb[4] + Eb[6]
            i1 = Eb[1] + Eb[3]
            k1 = Eb[5] + Eb[7]
            s00, s01 = a + b2, c + f
            s10, s11 = a + c, b2 + f
            s20, s21 = g0 + h0, i1 + k1
            is0 = d == 0
            is1 = d == 1
            S0 = jnp.where(is0, s00, jnp.where(is1, s10, s20))
            S1 = jnp.where(is0, s01, jnp.where(is1, s11, s21))
            f0 = M + _log_1_8(S0) - m0
            f1 = M + _log_1_8(S1) - m1
            oa_v[sl] = f0
            ob_v[sl] = f1
            plsc.store_scatter(f2v8_v, [idxv, zeros16], f0)
            plsc.store_scatter(f2v8_v, [idxv, ones16], f1)
            return c2

        lax.fori_loop(0, CHUNK // 16, grp, 0)

    def fire_scatters(g):
        slot = g % 2
        off = w * PW + g * CHUNK
        descs = [pltpu.async_copy(o8s[slot].at[pl.ds(j * 128, 128)],
                                  vacc_sp.at[vidx_a.at[g * NB + j]],
                                  sss[slot], add=True)
                 for j in range(NB)]
        for dsc in descs:
            dsc.wait()
        return [pltpu.async_copy(oas[slot], f2v_hbm.at[0, pl.ds(off, CHUNK)],
                                 sso[slot]),
                pltpu.async_copy(obs[slot], f2v_hbm.at[1, pl.ds(off, CHUNK)],
                                 sso[slot])]

    gd = {0: fire_gathers(0)}
    sd = {}
    for g in range(KCH):
        if g >= 1:
            for dsc in sd.pop(g - 1):
                dsc.wait()
        if g + 1 < KCH:
            gd[g + 1] = fire_gathers(g + 1)
        for dsc in gd.pop(g):
            dsc.wait()
        compute(g)
        sd[g] = fire_scatters(g)
    for dsc in sd.pop(KCH - 1):
        dsc.wait()

    plsc.subcore_barrier()
    pltpu.sync_copy(vacc_sp.at[pl.ds(sid * STRIPE, STRIPE)],
                    vaccp_hbm.at[cid, pl.ds(sid * STRIPE, STRIPE)])


def _k3_body(vb_hbm, f2v_hbm, fidx2_hbm, vidx2_hbm, evi_hbm, z8_hbm,
             v2f_hbm, faccp_hbm,
             facc_sp, fidx_a, vidx_a, e0b, e1b, f00, f01, f10, f11,
             g0b, g1b, va0, vb0, va1, vb1, e80, e81,
             sg0, sg1, ss0, ss1, so0, so1):
    cid = lax.axis_index("c")
    sid = lax.axis_index("s")
    pltpu.sync_copy(z8_hbm.at[pl.ds(sid * STRIPE, STRIPE)],
                    facc_sp.at[pl.ds(sid * STRIPE, STRIPE)])
    plsc.subcore_barrier()

    w = cid * NS + sid
    rows_pw = PW // 128
    pltpu.sync_copy(fidx2_hbm.at[pl.ds(w * rows_pw, rows_pw)], fidx_a)
    pltpu.sync_copy(vidx2_hbm.at[pl.ds(w * rows_pw, rows_pw)], vidx_a)

    iota16 = lax.iota(jnp.int32, 16)
    zeros16 = jnp.zeros((16,), jnp.int32)
    ones16 = jnp.full((16,), 1, jnp.int32)
    ebs = [e0b, e1b]
    f0s = [f00, f01]
    f1s = [f10, f11]
    gbs = [g0b, g1b]
    vas = [va0, va1]
    vbs = [vb0, vb1]
    e8s = [e80, e81]
    sgs = [sg0, sg1]
    sss = [ss0, ss1]
    sso = [so0, so1]

    def fire_gathers(g):
        slot = g % 2
        off = w * PW + g * CHUNK
        descs = [pltpu.async_copy(vb_hbm.at[vidx_a.at[g * NB + j]],
                                  gbs[slot].at[pl.ds(j * 128, 128)], sgs[slot])
                 for j in range(NB)]
        descs.append(pltpu.async_copy(f2v_hbm.at[0, pl.ds(off, CHUNK)],
                                      f0s[slot], sgs[slot]))
        descs.append(pltpu.async_copy(f2v_hbm.at[1, pl.ds(off, CHUNK)],
                                      f1s[slot], sgs[slot]))
        descs.append(pltpu.async_copy(evi_hbm.at[pl.ds(off, CHUNK)],
                                      ebs[slot], sgs[slot]))
        return descs

    def compute(g):
        slot = g % 2
        g_v, exp_v = gbs[slot], e8s[slot]
        f0_v, f1_v, evi_v = f0s[slot], f1s[slot], ebs[slot]
        va_v, vb_v = vas[slot], vbs[slot]

        def grp(i, c2):
            sl = pl.ds(i * 16, 16)
            idxv = iota16 + i * 16
            d = evi_v[sl]
            shift = 2 - d
            gg0 = plsc.load_gather(g_v, [idxv, zeros16])
            gg1 = plsc.load_gather(g_v, [idxv, ones16])
            v0 = gg0 - f0_v[sl]
            v1 = gg1 - f1_v[sl]
            va_v[sl] = v0
            vb_v[sl] = v1
            for b in range(8):
                bit = jnp.bitwise_and(
                    jnp.right_shift(jnp.full((16,), b, jnp.int32), shift), 1)
                tv = jnp.where(bit == 1, v1, v0)
                plsc.store_scatter(exp_v, [idxv, jnp.full((16,), b, jnp.int32)],
                                   tv)
            return c2

        lax.fori_loop(0, CHUNK // 16, grp, 0)

    def fire_scatters(g):
        slot = g % 2
        off = w * PW + g * CHUNK
        descs = [pltpu.async_copy(e8s[slot].at[pl.ds(j * 128, 128)],
                                  facc_sp.at[fidx_a.at[g * NB + j]],
                                  sss[slot], add=True)
                 for j in range(NB)]
        for dsc in descs:
            dsc.wait()
        return [pltpu.async_copy(vas[slot], v2f_hbm.at[0, pl.ds(off, CHUNK)],
                                 sso[slot]),
                pltpu.async_copy(vbs[slot], v2f_hbm.at[1, pl.ds(off, CHUNK)],
                                 sso[slot])]

    gd = {0: fire_gathers(0)}
    sd = {}
    for g in range(KCH):
        if g >= 1:
            for dsc in sd.pop(g - 1):
                dsc.wait()
        if g + 1 < KCH:
            gd[g + 1] = fire_gathers(g + 1)
        for dsc in gd.pop(g):
            dsc.wait()
        compute(g)
        sd[g] = fire_scatters(g)
    for dsc in sd.pop(KCH - 1):
        dsc.wait()

    plsc.subcore_barrier()
    pltpu.sync_copy(facc_sp.at[pl.ds(sid * STRIPE, STRIPE)],
                    faccp_hbm.at[cid, pl.ds(sid * STRIPE, STRIPE)])


def _k0_body(fbT_ref, o_ref):
    o_ref[...] = jnp.transpose(fbT_ref[...])


def _k2_body(vp_ref, soa_ref, vb8_ref):
    x0 = vp_ref[0, :, 0] + vp_ref[1, :, 0]
    x1 = vp_ref[0, :, 1] + vp_ref[1, :, 1]
    m = jnp.maximum(x0, x1)
    lse = m + jnp.log(jnp.exp(x0 - m) + jnp.exp(x1 - m))
    o0 = x0 - lse
    o1 = x1 - lse
    soa_ref[0, :] = o0
    soa_ref[1, :] = o1
    vb8_ref[...] = jnp.concatenate(
        [o0[:, None], o1[:, None], jnp.zeros((o0.shape[0], 6), o0.dtype)],
        axis=1)


def _k4_body(fp_ref, pot_ref, o_ref):
    x = fp_ref[0] + fp_ref[1] + jnp.transpose(pot_ref[...])
    m = jnp.max(x, axis=1, keepdims=True)
    lse = m + jnp.log(jnp.sum(jnp.exp(x - m), axis=1, keepdims=True))
    o_ref[...] = jnp.transpose(x - lse)


@jax.jit
def kernel(factor_potentials, factor_beliefs, var_beliefs,
           prv_varToFactor_messages, prv_factorToVar_messages,
           factorToVar_edge_index, edge_var_indices):
    del var_beliefs, prv_factorToVar_messages
    f32 = jnp.float32
    i32 = jnp.int32

    fac_idx = factorToVar_edge_index[0]
    var_idx = factorToVar_edge_index[1]
    evi = edge_var_indices[0]

    fbT_pad = jnp.zeros((8, R_PAD), f32).at[:, :F].set(
        factor_beliefs.reshape(F, 8).T)
    m01 = jnp.zeros((2, E_PAD), f32).at[:, :E].set(prv_varToFactor_messages.T)
    fidx_pad = jnp.full((E_PAD,), F, i32).at[:E].set(fac_idx)
    vidx_pad = jnp.full((E_PAD,), V, i32).at[:E].set(var_idx)
    evi_pad = jnp.zeros((E_PAD,), i32).at[:E].set(evi)
    fidx2 = fidx_pad.reshape(E_PAD // 128, 128)
    vidx2 = vidx_pad.reshape(E_PAD // 128, 128)
    z8 = jnp.zeros((R_PAD, 8), f32)
    potT = jnp.zeros((8, R_PAD), f32).at[:, :F].set(
        factor_potentials.reshape(F, 8).T)

    mesh = plsc.VectorSubcoreMesh(core_axis_name="c", subcore_axis_name="s")
    sc_params = pltpu.CompilerParams(needs_layout_passes=False,
                                     use_tc_tiling_on_sc=False)

    fb_pad = pl.pallas_call(
        _k0_body,
        out_shape=jax.ShapeDtypeStruct((R_PAD, 8), f32),
        grid=(R_PAD // RB,),
        in_specs=[pl.BlockSpec((8, RB), lambda i: (0, i))],
        out_specs=pl.BlockSpec((RB, 8), lambda i: (i, 0)),
    )(fbT_pad)

    k1 = pl.kernel(
        _k1_body,
        out_type=(jax.ShapeDtypeStruct((2, E_PAD), f32),
                  jax.ShapeDtypeStruct((NC, R_PAD, 8), f32)),
        mesh=mesh,
        compiler_params=sc_params,
        scratch_types=[
            pltpu.VMEM_SHARED((R_PAD, 8), f32),
            pltpu.VMEM((PW // 128, 128), i32),
            pltpu.VMEM((PW // 128, 128), i32),
            pltpu.VMEM((CHUNK,), i32),
            pltpu.VMEM((CHUNK,), i32),
            pltpu.VMEM((CHUNK,), f32),
            pltpu.VMEM((CHUNK,), f32),
            pltpu.VMEM((CHUNK,), f32),
            pltpu.VMEM((CHUNK,), f32),
            pltpu.VMEM((CHUNK, 8), f32),
            pltpu.VMEM((CHUNK, 8), f32),
            pltpu.VMEM((CHUNK,), f32),
            pltpu.VMEM((CHUNK,), f32),
            pltpu.VMEM((CHUNK,), f32),
            pltpu.VMEM((CHUNK,), f32),
            pltpu.VMEM((CHUNK, 8), f32),
            pltpu.VMEM((CHUNK, 8), f32),
            pltpu.SemaphoreType.DMA,
            pltpu.SemaphoreType.DMA,
            pltpu.SemaphoreType.DMA,
            pltpu.SemaphoreType.DMA,
            pltpu.SemaphoreType.DMA,
            pltpu.SemaphoreType.DMA,
        ],
    )
    f2v_soa, vaccp = k1(fb_pad, fidx2, vidx2, evi_pad, m01, z8)

    vb_soa, vb8 = pl.pallas_call(
        _k2_body,
        out_shape=(jax.ShapeDtypeStruct((2, R_PAD), f32),
                   jax.ShapeDtypeStruct((R_PAD, 8), f32)),
        grid=(R_PAD // RB,),
        in_specs=[pl.BlockSpec((NC, RB, 8), lambda i: (0, i, 0))],
        out_specs=(pl.BlockSpec((2, RB), lambda i: (0, i)),
                   pl.BlockSpec((RB, 8), lambda i: (i, 0))),
    )(vaccp)

    k3 = pl.kernel(
        _k3_body,
        out_type=(jax.ShapeDtypeStruct((2, E_PAD), f32),
                  jax.ShapeDtypeStruct((NC, R_PAD, 8), f32)),
        mesh=mesh,
        compiler_params=sc_params,
        scratch_types=[
            pltpu.VMEM_SHARED((R_PAD, 8), f32),
            pltpu.VMEM((PW // 128, 128), i32),
            pltpu.VMEM((PW // 128, 128), i32),
            pltpu.VMEM((CHUNK,), i32),
            pltpu.VMEM((CHUNK,), i32),
            pltpu.VMEM((CHUNK,), f32),
            pltpu.VMEM((CHUNK,), f32),
            pltpu.VMEM((CHUNK,), f32),
            pltpu.VMEM((CHUNK,), f32),
            pltpu.VMEM((CHUNK, 8), f32),
            pltpu.VMEM((CHUNK, 8), f32),
            pltpu.VMEM((CHUNK,), f32),
            pltpu.VMEM((CHUNK,), f32),
            pltpu.VMEM((CHUNK,), f32),
            pltpu.VMEM((CHUNK,), f32),
            pltpu.VMEM((CHUNK, 8), f32),
            pltpu.VMEM((CHUNK, 8), f32),
            pltpu.SemaphoreType.DMA,
            pltpu.SemaphoreType.DMA,
            pltpu.SemaphoreType.DMA,
            pltpu.SemaphoreType.DMA,
            pltpu.SemaphoreType.DMA,
            pltpu.SemaphoreType.DMA,
        ],
    )
    v2f_soa, faccp = k3(vb8, f2v_soa, fidx2, vidx2, evi_pad, z8)

    fb_soa = pl.pallas_call(
        _k4_body,
        out_shape=jax.ShapeDtypeStruct((8, R_PAD), f32),
        grid=(R_PAD // RB,),
        in_specs=[pl.BlockSpec((NC, RB, 8), lambda i: (0, i, 0)),
                  pl.BlockSpec((8, RB), lambda i: (0, i))],
        out_specs=pl.BlockSpec((8, RB), lambda i: (0, i)),
    )(faccp, potT)

    var_beliefs_new = vb_soa[:, :V].T
    factor_beliefs_new = fb_soa[:, :F].reshape(2, 2, 2, F).transpose(3, 0, 1, 2)
    factorToVar_messages = f2v_soa[:, :E].T
    varToFactor_messages = v2f_soa[:, :E].T
    return (var_beliefs_new, factor_beliefs_new, factorToVar_messages,
            varToFactor_messages)


# speedup vs baseline: 1.0150x; 1.0150x over previous
import jax
import jax.numpy as jnp
from jax import lax
from jax.experimental import pallas as pl
from jax.experimental.pallas import tpu as pltpu
from jax.experimental.pallas import tpu_sc as plsc

F = 100000
V = 100000
E = 300000

NC = 2
NS = 16
NW = NC * NS
CHUNK = 512
NB = CHUNK // 128
KCH = 20
PW = CHUNK * KCH
E_PAD = NW * PW
R_PAD = 100352
STRIPE = R_PAD // NS
RB = 1024

_LN2 = 0.6931471805599453


def _log_1_8(s):
    bits = lax.bitcast_convert_type(s, jnp.int32)
    e = jnp.right_shift(bits, 23) - 127
    mb = jnp.bitwise_or(jnp.bitwise_and(bits, 0x7FFFFF), 0x3F800000)
    m = lax.bitcast_convert_type(mb, jnp.float32)
    z = (m - 1.0) / (m + 1.0)
    z2 = z * z
    p = 2.0 * z * (1.0 + z2 * (1.0 / 3.0 + z2 * (0.2 + z2 * (1.0 / 7.0))))
    return e.astype(jnp.float32) * _LN2 + p


def _k1_body(fb_hbm, fidx2_hbm, vidx2_hbm, evi_hbm, m01_hbm, z8_hbm,
             f2v_hbm, vaccp_hbm,
             vacc_sp, fidx_a, vidx_a, e0b, e1b, m00, m01, m10, m11,
             fb0, fb1, oa0, ob0, oa1, ob1, o80, o81,
             sg0, sg1, ss0, ss1, so0, so1):
    cid = lax.axis_index("c")
    sid = lax.axis_index("s")
    pltpu.sync_copy(z8_hbm.at[pl.ds(sid * STRIPE, STRIPE)],
                    vacc_sp.at[pl.ds(sid * STRIPE, STRIPE)])
    pltpu.sync_copy(z8_hbm.at[pl.ds(0, CHUNK)], o80)
    pltpu.sync_copy(z8_hbm.at[pl.ds(0, CHUNK)], o81)
    plsc.subcore_barrier()

    w = cid * NS + sid
    rows_pw = PW // 128
    pltpu.sync_copy(fidx2_hbm.at[pl.ds(w * rows_pw, rows_pw)], fidx_a)
    pltpu.sync_copy(vidx2_hbm.at[pl.ds(w * rows_pw, rows_pw)], vidx_a)

    iota16 = lax.iota(jnp.int32, 16)
    zeros16 = jnp.zeros((16,), jnp.int32)
    ones16 = jnp.full((16,), 1, jnp.int32)
    ebs = [e0b, e1b]
    m0s = [m00, m01]
    m1s = [m10, m11]
    fbs = [fb0, fb1]
    oas = [oa0, oa1]
    obs = [ob0, ob1]
    o8s = [o80, o81]
    sgs = [sg0, sg1]
    sss = [ss0, ss1]
    sso = [so0, so1]

    def fire_gathers(g):
        slot = g % 2
        off = w * PW + g * CHUNK
        descs = [pltpu.async_copy(fb_hbm.at[fidx_a.at[g * NB + j]],
                                  fbs[slot].at[pl.ds(j * 128, 128)], sgs[slot])
                 for j in range(NB)]
        descs.append(pltpu.async_copy(m01_hbm.at[0, pl.ds(off, CHUNK)],
                                      m0s[slot], sgs[slot]))
        descs.append(pltpu.async_copy(m01_hbm.at[1, pl.ds(off, CHUNK)],
                                      m1s[slot], sgs[slot]))
        descs.append(pltpu.async_copy(evi_hbm.at[pl.ds(off, CHUNK)],
                                      ebs[slot], sgs[slot]))
        return descs

    def compute(g):
        slot = g % 2
        fbrows_v, f2v8_v = fbs[slot], o8s[slot]
        m0_v, m1_v, evi_v = m0s[slot], m1s[slot], ebs[slot]
        oa_v, ob_v = oas[slot], obs[slot]

        def grp(i, c2):
            sl = pl.ds(i * 16, 16)
            idxv = iota16 + i * 16
            d = evi_v[sl]
            m0 = m0_v[sl]
            m1 = m1_v[sl]
            B = [plsc.load_gather(fbrows_v,
                                  [idxv, jnp.full((16,), b, jnp.int32)])
                 for b in range(8)]
            M = jnp.maximum(jnp.maximum(jnp.maximum(B[0], B[1]),
                                        jnp.maximum(B[2], B[3])),
                            jnp.maximum(jnp.maximum(B[4], B[5]),
                                        jnp.maximum(B[6], B[7])))
            Eb = [jnp.exp(x - M) for x in B]
            a = Eb[0] + Eb[1]
            b2 = Eb[2] + Eb[3]
            c = Eb[4] + Eb[5]
            f = Eb[6] + Eb[7]
            g0 = Eb[0] + Eb[2]
            h0 = Eb[4] + Eb[6]
            i1 = Eb[1] + Eb[3]
            k1 = Eb[5] + Eb[7]
            s00, s01 = a + b2, c + f
            s10, s11 = a + c, b2 + f
            s20, s21 = g0 + h0, i1 + k1
            is0 = d == 0
            is1 = d == 1
            S0 = jnp.where(is0, s00, jnp.where(is1, s10, s20))
            S1 = jnp.where(is0, s01, jnp.where(is1, s11, s21))
            f0 = M + _log_1_8(S0) - m0
            f1 = M + _log_1_8(S1) - m1
            oa_v[sl] = f0
            ob_v[sl] = f1
            plsc.store_scatter(f2v8_v, [idxv, zeros16], f0)
            plsc.store_scatter(f2v8_v, [idxv, ones16], f1)
            return c2

        lax.fori_loop(0, CHUNK // 16, grp, 0)

    def fire_scatters(g):
        slot = g % 2
        off = w * PW + g * CHUNK
        descs = [pltpu.async_copy(o8s[slot].at[pl.ds(j * 128, 128)],
                                  vacc_sp.at[vidx_a.at[g * NB + j]],
                                  sss[slot], add=True)
                 for j in range(NB)]
        for dsc in descs:
            dsc.wait()
        return [pltpu.async_copy(oas[slot], f2v_hbm.at[0, pl.ds(off, CHUNK)],
                                 sso[slot]),
                pltpu.async_copy(obs[slot], f2v_hbm.at[1, pl.ds(off, CHUNK)],
                                 sso[slot])]

    gd = {0: fire_gathers(0)}
    sd = {}
    for g in range(KCH):
        if g >= 1:
            for dsc in sd.pop(g - 1):
                dsc.wait()
        if g + 1 < KCH:
            gd[g + 1] = fire_gathers(g + 1)
        for dsc in gd.pop(g):
            dsc.wait()
        compute(g)
        sd[g] = fire_scatters(g)
    for dsc in sd.pop(KCH - 1):
        dsc.wait()

    plsc.subcore_barrier()
    pltpu.sync_copy(vacc_sp.at[pl.ds(sid * STRIPE, STRIPE)],
                    vaccp_hbm.at[cid, pl.ds(sid * STRIPE, STRIPE)])


def _k3_body(vb_hbm, f2v_hbm, fidx2_hbm, vidx2_hbm, evi_hbm, z8_hbm,
             v2f_hbm, faccp_hbm,
             facc_sp, fidx_a, vidx_a, e0b, e1b, f00, f01, f10, f11,
             g0b, g1b, va0, vb0, va1, vb1, e80, e81,
             sg0, sg1, ss0, ss1, so0, so1):
    cid = lax.axis_index("c")
    sid = lax.axis_index("s")
    pltpu.sync_copy(z8_hbm.at[pl.ds(sid * STRIPE, STRIPE)],
                    facc_sp.at[pl.ds(sid * STRIPE, STRIPE)])
    plsc.subcore_barrier()

    w = cid * NS + sid
    rows_pw = PW // 128
    pltpu.sync_copy(fidx2_hbm.at[pl.ds(w * rows_pw, rows_pw)], fidx_a)
    pltpu.sync_copy(vidx2_hbm.at[pl.ds(w * rows_pw, rows_pw)], vidx_a)

    iota16 = lax.iota(jnp.int32, 16)
    zeros16 = jnp.zeros((16,), jnp.int32)
    ones16 = jnp.full((16,), 1, jnp.int32)
    ebs = [e0b, e1b]
    f0s = [f00, f01]
    f1s = [f10, f11]
    gbs = [g0b, g1b]
    vas = [va0, va1]
    vbs = [vb0, vb1]
    e8s = [e80, e81]
    sgs = [sg0, sg1]
    sss = [ss0, ss1]
    sso = [so0, so1]

    def fire_gathers(g):
        slot = g % 2
        off = w * PW + g * CHUNK
        descs = [pltpu.async_copy(vb_hbm.at[vidx_a.at[g * NB + j]],
                                  gbs[slot].at[pl.ds(j * 128, 128)], sgs[slot])
                 for j in range(NB)]
        descs.append(pltpu.async_copy(f2v_hbm.at[0, pl.ds(off, CHUNK)],
                                      f0s[slot], sgs[slot]))
        descs.append(pltpu.async_copy(f2v_hbm.at[1, pl.ds(off, CHUNK)],
                                      f1s[slot], sgs[slot]))
        descs.append(pltpu.async_copy(evi_hbm.at[pl.ds(off, CHUNK)],
                                      ebs[slot], sgs[slot]))
        return descs

    def compute(g):
        slot = g % 2
        g_v, exp_v = gbs[slot], e8s[slot]
        f0_v, f1_v, evi_v = f0s[slot], f1s[slot], ebs[slot]
        va_v, vb_v = vas[slot], vbs[slot]

        def grp(i, c2):
            sl = pl.ds(i * 16, 16)
            idxv = iota16 + i * 16
            d = evi_v[sl]
            shift = 2 - d
            gg0 = plsc.load_gather(g_v, [idxv, zeros16])
            gg1 = plsc.load_gather(g_v, [idxv, ones16])
            v0 = gg0 - f0_v[sl]
            v1 = gg1 - f1_v[sl]
            va_v[sl] = v0
            vb_v[sl] = v1
            for b in range(8):
                bit = jnp.bitwise_and(
                    jnp.right_shift(jnp.full((16,), b, jnp.int32), shift), 1)
                tv = jnp.where(bit == 1, v1, v0)
                plsc.store_scatter(exp_v, [idxv, jnp.full((16,), b, jnp.int32)],
                                   tv)
            return c2

        lax.fori_loop(0, CHUNK // 16, grp, 0)

    def fire_scatters(g):
        slot = g % 2
        off = w * PW + g * CHUNK
        descs = [pltpu.async_copy(e8s[slot].at[pl.ds(j * 128, 128)],
                                  facc_sp.at[fidx_a.at[g * NB + j]],
                                  sss[slot], add=True)
                 for j in range(NB)]
        for dsc in descs:
            dsc.wait()
        return [pltpu.async_copy(vas[slot], v2f_hbm.at[0, pl.ds(off, CHUNK)],
                                 sso[slot]),
                pltpu.async_copy(vbs[slot], v2f_hbm.at[1, pl.ds(off, CHUNK)],
                                 sso[slot])]

    gd = {0: fire_gathers(0)}
    sd = {}
    for g in range(KCH):
        if g >= 1:
            for dsc in sd.pop(g - 1):
                dsc.wait()
        if g + 1 < KCH:
            gd[g + 1] = fire_gathers(g + 1)
        for dsc in gd.pop(g):
            dsc.wait()
        compute(g)
        sd[g] = fire_scatters(g)
    for dsc in sd.pop(KCH - 1):
        dsc.wait()

    plsc.subcore_barrier()
    pltpu.sync_copy(facc_sp.at[pl.ds(sid * STRIPE, STRIPE)],
                    faccp_hbm.at[cid, pl.ds(sid * STRIPE, STRIPE)])


def _k2_body(vp_ref, soa_ref, vb8_ref):
    x0 = vp_ref[0, :, 0] + vp_ref[1, :, 0]
    x1 = vp_ref[0, :, 1] + vp_ref[1, :, 1]
    m = jnp.maximum(x0, x1)
    lse = m + jnp.log(jnp.exp(x0 - m) + jnp.exp(x1 - m))
    o0 = x0 - lse
    o1 = x1 - lse
    soa_ref[0, :] = o0
    soa_ref[1, :] = o1
    vb8_ref[...] = jnp.concatenate(
        [o0[:, None], o1[:, None], jnp.zeros((o0.shape[0], 6), o0.dtype)],
        axis=1)


def _k4_body(fp_ref, pot_ref, o_ref):
    x = fp_ref[0] + fp_ref[1] + jnp.transpose(pot_ref[...])
    m = jnp.max(x, axis=1, keepdims=True)
    lse = m + jnp.log(jnp.sum(jnp.exp(x - m), axis=1, keepdims=True))
    o_ref[...] = jnp.transpose(x - lse)


@jax.jit
def kernel(factor_potentials, factor_beliefs, var_beliefs,
           prv_varToFactor_messages, prv_factorToVar_messages,
           factorToVar_edge_index, edge_var_indices):
    del var_beliefs, prv_factorToVar_messages
    f32 = jnp.float32
    i32 = jnp.int32

    fac_idx = factorToVar_edge_index[0]
    var_idx = factorToVar_edge_index[1]
    evi = edge_var_indices[0]

    fb_pad = jnp.zeros((R_PAD, 8), f32).at[:F].set(factor_beliefs.reshape(F, 8))
    m01 = jnp.zeros((2, E_PAD), f32).at[:, :E].set(prv_varToFactor_messages.T)
    fidx_pad = jnp.full((E_PAD,), F, i32).at[:E].set(fac_idx)
    vidx_pad = jnp.full((E_PAD,), V, i32).at[:E].set(var_idx)
    evi_pad = jnp.zeros((E_PAD,), i32).at[:E].set(evi)
    fidx2 = fidx_pad.reshape(E_PAD // 128, 128)
    vidx2 = vidx_pad.reshape(E_PAD // 128, 128)
    z8 = jnp.zeros((R_PAD, 8), f32)
    potT = jnp.zeros((8, R_PAD), f32).at[:, :F].set(
        factor_potentials.reshape(F, 8).T)

    mesh = plsc.VectorSubcoreMesh(core_axis_name="c", subcore_axis_name="s")
    sc_params = pltpu.CompilerParams(needs_layout_passes=False,
                                     use_tc_tiling_on_sc=False)

    k1 = pl.kernel(
        _k1_body,
        out_type=(jax.ShapeDtypeStruct((2, E_PAD), f32),
                  jax.ShapeDtypeStruct((NC, R_PAD, 8), f32)),
        mesh=mesh,
        compiler_params=sc_params,
        scratch_types=[
            pltpu.VMEM_SHARED((R_PAD, 8), f32),
            pltpu.VMEM((PW // 128, 128), i32),
            pltpu.VMEM((PW // 128, 128), i32),
            pltpu.VMEM((CHUNK,), i32),
            pltpu.VMEM((CHUNK,), i32),
            pltpu.VMEM((CHUNK,), f32),
            pltpu.VMEM((CHUNK,), f32),
            pltpu.VMEM((CHUNK,), f32),
            pltpu.VMEM((CHUNK,), f32),
            pltpu.VMEM((CHUNK, 8), f32),
            pltpu.VMEM((CHUNK, 8), f32),
            pltpu.VMEM((CHUNK,), f32),
            pltpu.VMEM((CHUNK,), f32),
            pltpu.VMEM((CHUNK,), f32),
            pltpu.VMEM((CHUNK,), f32),
            pltpu.VMEM((CHUNK, 8), f32),
            pltpu.VMEM((CHUNK, 8), f32),
            pltpu.SemaphoreType.DMA,
            pltpu.SemaphoreType.DMA,
            pltpu.SemaphoreType.DMA,
            pltpu.SemaphoreType.DMA,
            pltpu.SemaphoreType.DMA,
            pltpu.SemaphoreType.DMA,
        ],
    )
    f2v_soa, vaccp = k1(fb_pad, fidx2, vidx2, evi_pad, m01, z8)

    vb_soa, vb8 = pl.pallas_call(
        _k2_body,
        out_shape=(jax.ShapeDtypeStruct((2, R_PAD), f32),
                   jax.ShapeDtypeStruct((R_PAD, 8), f32)),
        grid=(R_PAD // RB,),
        in_specs=[pl.BlockSpec((NC, RB, 8), lambda i: (0, i, 0))],
        out_specs=(pl.BlockSpec((2, RB), lambda i: (0, i)),
                   pl.BlockSpec((RB, 8), lambda i: (i, 0))),
    )(vaccp)

    k3 = pl.kernel(
        _k3_body,
        out_type=(jax.ShapeDtypeStruct((2, E_PAD), f32),
                  jax.ShapeDtypeStruct((NC, R_PAD, 8), f32)),
        mesh=mesh,
        compiler_params=sc_params,
        scratch_types=[
            pltpu.VMEM_SHARED((R_PAD, 8), f32),
            pltpu.VMEM((PW // 128, 128), i32),
            pltpu.VMEM((PW // 128, 128), i32),
            pltpu.VMEM((CHUNK,), i32),
            pltpu.VMEM((CHUNK,), i32),
            pltpu.VMEM((CHUNK,), f32),
            pltpu.VMEM((CHUNK,), f32),
            pltpu.VMEM((CHUNK,), f32),
            pltpu.VMEM((CHUNK,), f32),
            pltpu.VMEM((CHUNK, 8), f32),
            pltpu.VMEM((CHUNK, 8), f32),
            pltpu.VMEM((CHUNK,), f32),
            pltpu.VMEM((CHUNK,), f32),
            pltpu.VMEM((CHUNK,), f32),
            pltpu.VMEM((CHUNK,), f32),
            pltpu.VMEM((CHUNK, 8), f32),
            pltpu.VMEM((CHUNK, 8), f32),
            pltpu.SemaphoreType.DMA,
            pltpu.SemaphoreType.DMA,
            pltpu.SemaphoreType.DMA,
            pltpu.SemaphoreType.DMA,
            pltpu.SemaphoreType.DMA,
            pltpu.SemaphoreType.DMA,
        ],
    )
    v2f_soa, faccp = k3(vb8, f2v_soa, fidx2, vidx2, evi_pad, z8)

    fb_soa = pl.pallas_call(
        _k4_body,
        out_shape=jax.ShapeDtypeStruct((8, R_PAD), f32),
        grid=(R_PAD // RB,),
        in_specs=[pl.BlockSpec((NC, RB, 8), lambda i: (0, i, 0)),
                  pl.BlockSpec((8, RB), lambda i: (0, i))],
        out_specs=pl.BlockSpec((8, RB), lambda i: (0, i)),
    )(faccp, potT)

    var_beliefs_new = vb_soa[:, :V].T
    factor_beliefs_new = fb_soa[:, :F].reshape(2, 2, 2, F).transpose(3, 0, 1, 2)
    factorToVar_messages = f2v_soa[:, :E].T
    varToFactor_messages = v2f_soa[:, :E].T
    return (var_beliefs_new, factor_beliefs_new, factorToVar_messages,
            varToFactor_messages)


# speedup vs baseline: 1.0153x; 1.0003x over previous
import jax
import jax.numpy as jnp
from jax import lax
from jax.experimental import pallas as pl
from jax.experimental.pallas import tpu as pltpu
from jax.experimental.pallas import tpu_sc as plsc

F = 100000
V = 100000
E = 300000

NC = 2
NS = 16
NW = NC * NS
CHUNK = 512
NB = CHUNK // 128
KCH = 20
PW = CHUNK * KCH
E_PAD = NW * PW
R_PAD = 100352
STRIPE = R_PAD // NS
RB = 1024

_LN2 = 0.6931471805599453


def _log_1_8(s):
    bits = lax.bitcast_convert_type(s, jnp.int32)
    e = jnp.right_shift(bits, 23) - 127
    mb = jnp.bitwise_or(jnp.bitwise_and(bits, 0x7FFFFF), 0x3F800000)
    m = lax.bitcast_convert_type(mb, jnp.float32)
    z = (m - 1.0) / (m + 1.0)
    z2 = z * z
    p = 2.0 * z * (1.0 + z2 * (1.0 / 3.0 + z2 * (0.2 + z2 * (1.0 / 7.0))))
    return e.astype(jnp.float32) * _LN2 + p


def _k1_body(fb_hbm, fidx2_hbm, vidx2_hbm, evi_hbm, m01_hbm, z8_hbm,
             f2v_hbm, vaccp_hbm,
             vacc_sp, fidx_a, vidx_a, e0b, e1b, m00, m01, m10, m11,
             fb0, fb1, oa0, ob0, oa1, ob1, o80, o81,
             sg0, sg1, ss0, ss1, so0, so1):
    cid = lax.axis_index("c")
    sid = lax.axis_index("s")
    pltpu.sync_copy(z8_hbm.at[pl.ds(sid * STRIPE, STRIPE)],
                    vacc_sp.at[pl.ds(sid * STRIPE, STRIPE)])
    pltpu.sync_copy(z8_hbm.at[pl.ds(0, CHUNK)], o80)
    pltpu.sync_copy(z8_hbm.at[pl.ds(0, CHUNK)], o81)
    plsc.subcore_barrier()

    w = cid * NS + sid
    rows_pw = PW // 128
    pltpu.sync_copy(fidx2_hbm.at[pl.ds(w * rows_pw, rows_pw)], fidx_a)
    pltpu.sync_copy(vidx2_hbm.at[pl.ds(w * rows_pw, rows_pw)], vidx_a)

    iota16 = lax.iota(jnp.int32, 16)
    zeros16 = jnp.zeros((16,), jnp.int32)
    ones16 = jnp.full((16,), 1, jnp.int32)
    ebs = [e0b, e1b]
    m0s = [m00, m01]
    m1s = [m10, m11]
    fbs = [fb0, fb1]
    oas = [oa0, oa1]
    obs = [ob0, ob1]
    o8s = [o80, o81]
    sgs = [sg0, sg1]
    sss = [ss0, ss1]
    sso = [so0, so1]

    def fire_gathers(g):
        slot = g % 2
        off = w * PW + g * CHUNK
        descs = [pltpu.async_copy(fb_hbm.at[fidx_a.at[g * NB + j]],
                                  fbs[slot].at[pl.ds(j * 128, 128)], sgs[slot])
                 for j in range(NB)]
        descs.append(pltpu.async_copy(m01_hbm.at[0, pl.ds(off, CHUNK)],
                                      m0s[slot], sgs[slot]))
        descs.append(pltpu.async_copy(m01_hbm.at[1, pl.ds(off, CHUNK)],
                                      m1s[slot], sgs[slot]))
        descs.append(pltpu.async_copy(evi_hbm.at[pl.ds(off, CHUNK)],
                                      ebs[slot], sgs[slot]))
        return descs

    def compute(g):
        slot = g % 2
        fbrows_v, f2v8_v = fbs[slot], o8s[slot]
        m0_v, m1_v, evi_v = m0s[slot], m1s[slot], ebs[slot]
        oa_v, ob_v = oas[slot], obs[slot]

        def grp(i, c2):
            sl = pl.ds(i * 16, 16)
            idxv = iota16 + i * 16
            d = evi_v[sl]
            m0 = m0_v[sl]
            m1 = m1_v[sl]
            B = [plsc.load_gather(fbrows_v,
                                  [idxv, jnp.full((16,), b, jnp.int32)])
                 for b in range(8)]
            M = jnp.maximum(jnp.maximum(jnp.maximum(B[0], B[1]),
                                        jnp.maximum(B[2], B[3])),
                            jnp.maximum(jnp.maximum(B[4], B[5]),
                                        jnp.maximum(B[6], B[7])))
            Eb = [jnp.exp(x - M) for x in B]
            a = Eb[0] + Eb[1]
            b2 = Eb[2] + Eb[3]
            c = Eb[4] + Eb[5]
            f = Eb[6] + Eb[7]
            g0 = Eb[0] + Eb[2]
            h0 = Eb[4] + Eb[6]
            i1 = Eb[1] + Eb[3]
            k1 = Eb[5] + Eb[7]
            s00, s01 = a + b2, c + f
            s10, s11 = a + c, b2 + f
            s20, s21 = g0 + h0, i1 + k1
            is0 = d == 0
            is1 = d == 1
            S0 = jnp.where(is0, s00, jnp.where(is1, s10, s20))
            S1 = jnp.where(is0, s01, jnp.where(is1, s11, s21))
            f0 = M + _log_1_8(S0) - m0
            f1 = M + _log_1_8(S1) - m1
            oa_v[sl] = f0
            ob_v[sl] = f1
            plsc.store_scatter(f2v8_v, [idxv, zeros16], f0)
            plsc.store_scatter(f2v8_v, [idxv, ones16], f1)
            return c2

        lax.fori_loop(0, CHUNK // 16, grp, 0)

    def fire_scatters(g):
        slot = g % 2
        off = w * PW + g * CHUNK
        descs = [pltpu.async_copy(o8s[slot].at[pl.ds(j * 128, 128)],
                                  vacc_sp.at[vidx_a.at[g * NB + j]],
                                  sss[slot], add=True)
                 for j in range(NB)]
        descs.append(
            pltpu.async_copy(oas[slot], f2v_hbm.at[0, pl.ds(off, CHUNK)],
                             sso[slot]))
        descs.append(
            pltpu.async_copy(obs[slot], f2v_hbm.at[1, pl.ds(off, CHUNK)],
                             sso[slot]))
        return descs

    gd = {0: fire_gathers(0)}
    sd = {}
    for g in range(KCH):
        if g >= 1:
            for dsc in sd.pop(g - 1):
                dsc.wait()
        if g + 1 < KCH:
            gd[g + 1] = fire_gathers(g + 1)
        for dsc in gd.pop(g):
            dsc.wait()
        compute(g)
        sd[g] = fire_scatters(g)
    for dsc in sd.pop(KCH - 1):
        dsc.wait()

    plsc.subcore_barrier()
    pltpu.sync_copy(vacc_sp.at[pl.ds(sid * STRIPE, STRIPE)],
                    vaccp_hbm.at[cid, pl.ds(sid * STRIPE, STRIPE)])


def _k3_body(vb_hbm, f2v_hbm, fidx2_hbm, vidx2_hbm, evi_hbm, z8_hbm,
             v2f_hbm, faccp_hbm,
             facc_sp, fidx_a, vidx_a, e0b, e1b, f00, f01, f10, f11,
             g0b, g1b, va0, vb0, va1, vb1, e80, e81,
             sg0, sg1, ss0, ss1, so0, so1):
    cid = lax.axis_index("c")
    sid = lax.axis_index("s")
    pltpu.sync_copy(z8_hbm.at[pl.ds(sid * STRIPE, STRIPE)],
                    facc_sp.at[pl.ds(sid * STRIPE, STRIPE)])
    plsc.subcore_barrier()

    w = cid * NS + sid
    rows_pw = PW // 128
    pltpu.sync_copy(fidx2_hbm.at[pl.ds(w * rows_pw, rows_pw)], fidx_a)
    pltpu.sync_copy(vidx2_hbm.at[pl.ds(w * rows_pw, rows_pw)], vidx_a)

    iota16 = lax.iota(jnp.int32, 16)
    zeros16 = jnp.zeros((16,), jnp.int32)
    ones16 = jnp.full((16,), 1, jnp.int32)
    ebs = [e0b, e1b]
    f0s = [f00, f01]
    f1s = [f10, f11]
    gbs = [g0b, g1b]
    vas = [va0, va1]
    vbs = [vb0, vb1]
    e8s = [e80, e81]
    sgs = [sg0, sg1]
    sss = [ss0, ss1]
    sso = [so0, so1]

    def fire_gathers(g):
        slot = g % 2
        off = w * PW + g * CHUNK
        descs = [pltpu.async_copy(vb_hbm.at[vidx_a.at[g * NB + j]],
                                  gbs[slot].at[pl.ds(j * 128, 128)], sgs[slot])
                 for j in range(NB)]
        descs.append(pltpu.async_copy(f2v_hbm.at[0, pl.ds(off, CHUNK)],
                                      f0s[slot], sgs[slot]))
        descs.append(pltpu.async_copy(f2v_hbm.at[1, pl.ds(off, CHUNK)],
                                      f1s[slot], sgs[slot]))
        descs.append(pltpu.async_copy(evi_hbm.at[pl.ds(off, CHUNK)],
                                      ebs[slot], sgs[slot]))
        return descs

    def compute(g):
        slot = g % 2
        g_v, exp_v = gbs[slot], e8s[slot]
        f0_v, f1_v, evi_v = f0s[slot], f1s[slot], ebs[slot]
        va_v, vb_v = vas[slot], vbs[slot]

        def grp(i, c2):
            sl = pl.ds(i * 16, 16)
            idxv = iota16 + i * 16
            d = evi_v[sl]
            shift = 2 - d
            gg0 = plsc.load_gather(g_v, [idxv, zeros16])
            gg1 = plsc.load_gather(g_v, [idxv, ones16])
            v0 = gg0 - f0_v[sl]
            v1 = gg1 - f1_v[sl]
            va_v[sl] = v0
            vb_v[sl] = v1
            for b in range(8):
                bit = jnp.bitwise_and(
                    jnp.right_shift(jnp.full((16,), b, jnp.int32), shift), 1)
                tv = jnp.where(bit == 1, v1, v0)
                plsc.store_scatter(exp_v, [idxv, jnp.full((16,), b, jnp.int32)],
                                   tv)
            return c2

        lax.fori_loop(0, CHUNK // 16, grp, 0)

    def fire_scatters(g):
        slot = g % 2
        off = w * PW + g * CHUNK
        descs = [pltpu.async_copy(e8s[slot].at[pl.ds(j * 128, 128)],
                                  facc_sp.at[fidx_a.at[g * NB + j]],
                                  sss[slot], add=True)
                 for j in range(NB)]
        descs.append(
            pltpu.async_copy(vas[slot], v2f_hbm.at[0, pl.ds(off, CHUNK)],
                             sso[slot]))
        descs.append(
            pltpu.async_copy(vbs[slot], v2f_hbm.at[1, pl.ds(off, CHUNK)],
                             sso[slot]))
        return descs

    gd = {0: fire_gathers(0)}
    sd = {}
    for g in range(KCH):
        if g >= 1:
            for dsc in sd.pop(g - 1):
                dsc.wait()
        if g + 1 < KCH:
            gd[g + 1] = fire_gathers(g + 1)
        for dsc in gd.pop(g):
            dsc.wait()
        compute(g)
        sd[g] = fire_scatters(g)
    for dsc in sd.pop(KCH - 1):
        dsc.wait()

    plsc.subcore_barrier()
    pltpu.sync_copy(facc_sp.at[pl.ds(sid * STRIPE, STRIPE)],
                    faccp_hbm.at[cid, pl.ds(sid * STRIPE, STRIPE)])


def _k2_body(vp_ref, soa_ref, vb8_ref):
    x0 = vp_ref[0, :, 0] + vp_ref[1, :, 0]
    x1 = vp_ref[0, :, 1] + vp_ref[1, :, 1]
    m = jnp.maximum(x0, x1)
    lse = m + jnp.log(jnp.exp(x0 - m) + jnp.exp(x1 - m))
    o0 = x0 - lse
    o1 = x1 - lse
    soa_ref[0, :] = o0
    soa_ref[1, :] = o1
    vb8_ref[...] = jnp.concatenate(
        [o0[:, None], o1[:, None], jnp.zeros((o0.shape[0], 6), o0.dtype)],
        axis=1)


def _k4_body(fp_ref, pot_ref, o_ref):
    x = fp_ref[0] + fp_ref[1] + jnp.transpose(pot_ref[...])
    m = jnp.max(x, axis=1, keepdims=True)
    lse = m + jnp.log(jnp.sum(jnp.exp(x - m), axis=1, keepdims=True))
    o_ref[...] = jnp.transpose(x - lse)


@jax.jit
def kernel(factor_potentials, factor_beliefs, var_beliefs,
           prv_varToFactor_messages, prv_factorToVar_messages,
           factorToVar_edge_index, edge_var_indices):
    del var_beliefs, prv_factorToVar_messages
    f32 = jnp.float32
    i32 = jnp.int32

    fac_idx = factorToVar_edge_index[0]
    var_idx = factorToVar_edge_index[1]
    evi = edge_var_indices[0]

    fb_pad = jnp.zeros((R_PAD, 8), f32).at[:F].set(factor_beliefs.reshape(F, 8))
    m01 = jnp.zeros((2, E_PAD), f32).at[:, :E].set(prv_varToFactor_messages.T)
    fidx_pad = jnp.full((E_PAD,), F, i32).at[:E].set(fac_idx)
    vidx_pad = jnp.full((E_PAD,), V, i32).at[:E].set(var_idx)
    evi_pad = jnp.zeros((E_PAD,), i32).at[:E].set(evi)
    fidx2 = fidx_pad.reshape(E_PAD // 128, 128)
    vidx2 = vidx_pad.reshape(E_PAD // 128, 128)
    z8 = jnp.zeros((R_PAD, 8), f32)
    potT = jnp.zeros((8, R_PAD), f32).at[:, :F].set(
        factor_potentials.reshape(F, 8).T)

    mesh = plsc.VectorSubcoreMesh(core_axis_name="c", subcore_axis_name="s")
    sc_params = pltpu.CompilerParams(needs_layout_passes=False,
                                     use_tc_tiling_on_sc=False)

    k1 = pl.kernel(
        _k1_body,
        out_type=(jax.ShapeDtypeStruct((2, E_PAD), f32),
                  jax.ShapeDtypeStruct((NC, R_PAD, 8), f32)),
        mesh=mesh,
        compiler_params=sc_params,
        scratch_types=[
            pltpu.VMEM_SHARED((R_PAD, 8), f32),
            pltpu.VMEM((PW // 128, 128), i32),
            pltpu.VMEM((PW // 128, 128), i32),
            pltpu.VMEM((CHUNK,), i32),
            pltpu.VMEM((CHUNK,), i32),
            pltpu.VMEM((CHUNK,), f32),
            pltpu.VMEM((CHUNK,), f32),
            pltpu.VMEM((CHUNK,), f32),
            pltpu.VMEM((CHUNK,), f32),
            pltpu.VMEM((CHUNK, 8), f32),
            pltpu.VMEM((CHUNK, 8), f32),
            pltpu.VMEM((CHUNK,), f32),
            pltpu.VMEM((CHUNK,), f32),
            pltpu.VMEM((CHUNK,), f32),
            pltpu.VMEM((CHUNK,), f32),
            pltpu.VMEM((CHUNK, 8), f32),
            pltpu.VMEM((CHUNK, 8), f32),
            pltpu.SemaphoreType.DMA,
            pltpu.SemaphoreType.DMA,
            pltpu.SemaphoreType.DMA,
            pltpu.SemaphoreType.DMA,
            pltpu.SemaphoreType.DMA,
            pltpu.SemaphoreType.DMA,
        ],
    )
    f2v_soa, vaccp = k1(fb_pad, fidx2, vidx2, evi_pad, m01, z8)

    vb_soa, vb8 = pl.pallas_call(
        _k2_body,
        out_shape=(jax.ShapeDtypeStruct((2, R_PAD), f32),
                   jax.ShapeDtypeStruct((R_PAD, 8), f32)),
        grid=(R_PAD // RB,),
        in_specs=[pl.BlockSpec((NC, RB, 8), lambda i: (0, i, 0))],
        out_specs=(pl.BlockSpec((2, RB), lambda i: (0, i)),
                   pl.BlockSpec((RB, 8), lambda i: (i, 0))),
    )(vaccp)

    k3 = pl.kernel(
        _k3_body,
        out_type=(jax.ShapeDtypeStruct((2, E_PAD), f32),
                  jax.ShapeDtypeStruct((NC, R_PAD, 8), f32)),
        mesh=mesh,
        compiler_params=sc_params,
        scratch_types=[
            pltpu.VMEM_SHARED((R_PAD, 8), f32),
            pltpu.VMEM((PW // 128, 128), i32),
            pltpu.VMEM((PW // 128, 128), i32),
            pltpu.VMEM((CHUNK,), i32),
            pltpu.VMEM((CHUNK,), i32),
            pltpu.VMEM((CHUNK,), f32),
            pltpu.VMEM((CHUNK,), f32),
            pltpu.VMEM((CHUNK,), f32),
            pltpu.VMEM((CHUNK,), f32),
            pltpu.VMEM((CHUNK, 8), f32),
            pltpu.VMEM((CHUNK, 8), f32),
            pltpu.VMEM((CHUNK,), f32),
            pltpu.VMEM((CHUNK,), f32),
            pltpu.VMEM((CHUNK,), f32),
            pltpu.VMEM((CHUNK,), f32),
            pltpu.VMEM((CHUNK, 8), f32),
            pltpu.VMEM((CHUNK, 8), f32),
            pltpu.SemaphoreType.DMA,
            pltpu.SemaphoreType.DMA,
            pltpu.SemaphoreType.DMA,
            pltpu.SemaphoreType.DMA,
            pltpu.SemaphoreType.DMA,
            pltpu.SemaphoreType.DMA,
        ],
    )
    v2f_soa, faccp = k3(vb8, f2v_soa, fidx2, vidx2, evi_pad, z8)

    fb_soa = pl.pallas_call(
        _k4_body,
        out_shape=jax.ShapeDtypeStruct((8, R_PAD), f32),
        grid=(R_PAD // RB,),
        in_specs=[pl.BlockSpec((NC, RB, 8), lambda i: (0, i, 0)),
                  pl.BlockSpec((8, RB), lambda i: (0, i))],
        out_specs=pl.BlockSpec((8, RB), lambda i: (0, i)),
    )(faccp, potT)

    var_beliefs_new = vb_soa[:, :V].T
    factor_beliefs_new = fb_soa[:, :F].reshape(2, 2, 2, F).transpose(3, 0, 1, 2)
    factorToVar_messages = f2v_soa[:, :E].T
    varToFactor_messages = v2f_soa[:, :E].T
    return (var_beliefs_new, factor_beliefs_new, factorToVar_messages,
            varToFactor_messages)
